# Initial kernel scaffold; baseline (speedup 1.0000x reference)
import jax
import jax.numpy as jnp
from jax.experimental import pallas as pl


def kernel(scores):
    raise NotImplementedError("write your pallas kernel here")


# speedup vs baseline: 6.3508x; 6.3508x over previous
import jax
import jax.numpy as jnp
from jax.experimental import pallas as pl

_K = 512
_N = 16384
_ROWS_PER_BLOCK = 8


def _topk_mask_body(x_ref, o_ref):
    x = x_ref[...]
    i = jax.lax.bitcast_convert_type(x, jnp.int32)
    v = i ^ ((i >> 31) & jnp.int32(0x7FFFFFFF))
    r = x.shape[0]
    lo0 = jnp.full((r, 1), -2147483648, jnp.int32)
    hi0 = jnp.full((r, 1), 2147483647, jnp.int32)

    def body(_, carry):
        lo, hi = carry
        mid = (lo & hi) + ((lo ^ hi) >> 1)
        cnt = jnp.sum((v >= mid).astype(jnp.int32), axis=1, keepdims=True)
        ge = cnt >= _K
        return jnp.where(ge, mid, lo), jnp.where(ge, hi, mid)

    lo, _ = jax.lax.fori_loop(0, 32, body, (lo0, hi0))
    o_ref[...] = (v >= lo).astype(jnp.float32)


def kernel(scores):
    bsz, nmax, _, ens = scores.shape
    s = jnp.transpose(scores, (0, 3, 1, 2)).reshape(bsz * ens, nmax * nmax)
    out = pl.pallas_call(
        _topk_mask_body,
        grid=(s.shape[0] // _ROWS_PER_BLOCK,),
        in_specs=[pl.BlockSpec((_ROWS_PER_BLOCK, _N), lambda r: (r, 0))],
        out_specs=pl.BlockSpec((_ROWS_PER_BLOCK, _N), lambda r: (r, 0)),
        out_shape=jax.ShapeDtypeStruct(s.shape, jnp.float32),
    )(s)
    out = out.reshape(bsz, ens, nmax, nmax)
    return jnp.transpose(out, (0, 2, 3, 1))
